# baseline (speedup 1.0000x reference)
import functools

import jax
import jax.numpy as jnp
from jax import lax
from jax.experimental import pallas as pl
from jax.experimental.pallas import tpu as pltpu
from jax.experimental.pallas import tpu_sc as plsc

CODEBOOK = 1024
FEATURES = 256
BATCH = 8
SEQ = 576
ROWS = BATCH * SEQ
BLOCK_ROWS = SEQ
NUM_BLOCKS = ROWS // BLOCK_ROWS


def _tc_body(q_ref, kt_ref, idx_ref, loss_ref, perp_ref, cnt_ref, l2k_ref):
    i = pl.program_id(0)
    qb = q_ref[...]
    kt = kt_ref[...]

    @pl.when(i == 0)
    def _prep():
        l2k_ref[...] = jnp.sum(kt * kt, axis=1, keepdims=True)

    sim_t = lax.dot_general(kt, qb, (((1,), (1,)), ((), ())),
                            preferred_element_type=jnp.float32)
    negdist = 2.0 * sim_t - l2k_ref[...]
    mx = jnp.max(negdist, axis=0, keepdims=True)
    eq = negdist == mx
    code_iota = lax.broadcasted_iota(
        jnp.int32, (CODEBOOK, BLOCK_ROWS), 0).astype(jnp.float32)
    cand = jnp.where(eq, code_iota, float(CODEBOOK))
    idx = jnp.min(cand, axis=0, keepdims=True).astype(jnp.int32)
    idx_ref[0, :, :] = idx

    eq_f = jnp.where(eq, 1.0, 0.0)
    cnt = lax.dot_general(eq_f, jnp.ones((BLOCK_ROWS, 1), jnp.float32),
                          (((1,), (0,)), ((), ())),
                          preferred_element_type=jnp.float32)
    block_loss = jnp.sum(qb * qb) - jnp.sum(mx)

    @pl.when(i == 0)
    def _init():
        cnt_ref[...] = cnt
        loss_ref[0, 0] = block_loss

    @pl.when(i > 0)
    def _acc():
        cnt_ref[...] += cnt
        loss_ref[0, 0] += block_loss

    @pl.when(i == NUM_BLOCKS - 1)
    def _final():
        loss_ref[0, 0] = loss_ref[0, 0] / float(ROWS * FEATURES)
        mean = cnt_ref[...] * (1.0 / ROWS)
        ent = jnp.sum(mean * jnp.log(mean + 1e-10))
        perp_ref[0, 0] = jnp.exp(-ent)


def _distance_argmin(q, emb):
    return pl.pallas_call(
        _tc_body,
        grid=(NUM_BLOCKS,),
        in_specs=[
            pl.BlockSpec((BLOCK_ROWS, FEATURES), lambda i: (i, 0)),
            pl.BlockSpec((CODEBOOK, FEATURES), lambda i: (0, 0)),
        ],
        out_specs=[
            pl.BlockSpec((1, 1, BLOCK_ROWS), lambda i: (i, 0, 0)),
            pl.BlockSpec(memory_space=pltpu.SMEM),
            pl.BlockSpec(memory_space=pltpu.SMEM),
        ],
        out_shape=[
            jax.ShapeDtypeStruct((NUM_BLOCKS, 1, BLOCK_ROWS), jnp.int32),
            jax.ShapeDtypeStruct((1, 1), jnp.float32),
            jax.ShapeDtypeStruct((1, 1), jnp.float32),
        ],
        scratch_shapes=[pltpu.VMEM((CODEBOOK, 1), jnp.float32),
                        pltpu.VMEM((CODEBOOK, 1), jnp.float32)],
        compiler_params=pltpu.CompilerParams(
            dimension_semantics=("arbitrary",)),
    )(q, emb)


def _sc_gather(emb, idx3):
    info = plsc.get_sparse_core_info()
    nw = info.num_cores * info.num_subcores
    per_w = ROWS // nw
    half = per_w // 2
    w_per_b = SEQ // per_w
    mesh = plsc.VectorSubcoreMesh(core_axis_name="c", subcore_axis_name="s")

    @functools.partial(
        pl.kernel, mesh=mesh,
        out_type=jax.ShapeDtypeStruct((BATCH, SEQ, FEATURES), jnp.float32),
        scratch_types=[
            pltpu.VMEM((SEQ,), jnp.int32),
            pltpu.VMEM((half, FEATURES), jnp.float32),
            pltpu.VMEM((half, FEATURES), jnp.float32),
            pltpu.SemaphoreType.DMA,
            pltpu.SemaphoreType.DMA,
        ],
    )
    def gather(emb_hbm, idx_hbm, out_hbm, idxrow, rows0, rows1, sem, osem):
        wid = lax.axis_index("s") * info.num_cores + lax.axis_index("c")
        b = wid // w_per_b
        off = (wid % w_per_b) * per_w
        pltpu.sync_copy(idx_hbm.at[b, 0], idxrow)
        cp0 = pltpu.async_copy(emb_hbm.at[idxrow.at[pl.ds(off, half)]],
                               rows0, sem)
        cp1 = pltpu.async_copy(emb_hbm.at[idxrow.at[pl.ds(off + half, half)]],
                               rows1, sem)
        cp0.wait()
        out0 = pltpu.async_copy(rows0, out_hbm.at[b, pl.ds(off, half)], osem)
        cp1.wait()
        out1 = pltpu.async_copy(rows1, out_hbm.at[b, pl.ds(off + half, half)],
                                osem)
        out0.wait()
        out1.wait()

    return gather(emb, idx3)


def kernel(x, emb):
    b, n, d = x.shape
    q = x.reshape(b * n, d)
    idx3, loss11, perp11 = _distance_argmin(q, emb)
    quantized = _sc_gather(emb, idx3)
    loss = loss11.reshape(())
    perp = perp11.reshape((1,))
    return quantized, loss, idx3, perp


# speedup vs baseline: 1.0053x; 1.0053x over previous
import functools

import jax
import jax.numpy as jnp
from jax import lax
from jax.experimental import pallas as pl
from jax.experimental.pallas import tpu as pltpu
from jax.experimental.pallas import tpu_sc as plsc

CODEBOOK = 1024
FEATURES = 256
BATCH = 8
SEQ = 576
ROWS = BATCH * SEQ
BLOCK_ROWS = SEQ
NUM_BLOCKS = ROWS // BLOCK_ROWS


def _tc_body(q_ref, kt_hbm, idx_ref, loss_ref, perp_ref, cnt_ref, l2k_ref,
             kt_vmem, ksem):
    i = pl.program_id(0)
    qb = q_ref[...]

    @pl.when(i == 0)
    def _prep():
        pltpu.make_async_copy(kt_hbm, kt_vmem, ksem).start()
        pltpu.make_async_copy(kt_hbm, kt_vmem, ksem).wait()
        ktp = kt_vmem[...]
        l2k_ref[...] = jnp.sum(ktp * ktp, axis=1, keepdims=True)
    kt = kt_vmem[...]

    sim_t = lax.dot_general(kt, qb, (((1,), (1,)), ((), ())),
                            preferred_element_type=jnp.float32)
    negdist = 2.0 * sim_t - l2k_ref[...]
    mx = jnp.max(negdist, axis=0, keepdims=True)
    eq = negdist == mx
    code_iota = lax.broadcasted_iota(
        jnp.int32, (CODEBOOK, BLOCK_ROWS), 0).astype(jnp.float32)
    cand = jnp.where(eq, code_iota, float(CODEBOOK))
    idx = jnp.min(cand, axis=0, keepdims=True).astype(jnp.int32)
    idx_ref[0, :, :] = idx

    eq_f = jnp.where(eq, 1.0, 0.0)
    cnt = lax.dot_general(eq_f, jnp.ones((BLOCK_ROWS, 1), jnp.float32),
                          (((1,), (0,)), ((), ())),
                          preferred_element_type=jnp.float32)
    block_loss = jnp.sum(qb * qb) - jnp.sum(mx)

    @pl.when(i == 0)
    def _init():
        cnt_ref[...] = cnt
        loss_ref[0, 0] = block_loss

    @pl.when(i > 0)
    def _acc():
        cnt_ref[...] += cnt
        loss_ref[0, 0] += block_loss

    @pl.when(i == NUM_BLOCKS - 1)
    def _final():
        loss_ref[0, 0] = loss_ref[0, 0] / float(ROWS * FEATURES)
        mean = cnt_ref[...] * (1.0 / ROWS)
        ent = jnp.sum(mean * jnp.log(mean + 1e-10))
        perp_ref[0, 0] = jnp.exp(-ent)


def _distance_argmin(q, emb):
    return pl.pallas_call(
        _tc_body,
        grid=(NUM_BLOCKS,),
        in_specs=[
            pl.BlockSpec((BLOCK_ROWS, FEATURES), lambda i: (i, 0)),
            pl.BlockSpec(memory_space=pltpu.MemorySpace.HBM),
        ],
        out_specs=[
            pl.BlockSpec((1, 1, BLOCK_ROWS), lambda i: (i, 0, 0)),
            pl.BlockSpec(memory_space=pltpu.SMEM),
            pl.BlockSpec(memory_space=pltpu.SMEM),
        ],
        out_shape=[
            jax.ShapeDtypeStruct((NUM_BLOCKS, 1, BLOCK_ROWS), jnp.int32),
            jax.ShapeDtypeStruct((1, 1), jnp.float32),
            jax.ShapeDtypeStruct((1, 1), jnp.float32),
        ],
        scratch_shapes=[pltpu.VMEM((CODEBOOK, 1), jnp.float32),
                        pltpu.VMEM((CODEBOOK, 1), jnp.float32),
                        pltpu.VMEM((CODEBOOK, FEATURES), jnp.float32),
                        pltpu.SemaphoreType.DMA],
        compiler_params=pltpu.CompilerParams(
            dimension_semantics=("arbitrary",)),
    )(q, emb)


def _sc_gather(emb, idx3):
    info = plsc.get_sparse_core_info()
    nw = info.num_cores * info.num_subcores
    per_w = ROWS // nw
    half = per_w // 2
    w_per_b = SEQ // per_w
    mesh = plsc.VectorSubcoreMesh(core_axis_name="c", subcore_axis_name="s")

    @functools.partial(
        pl.kernel, mesh=mesh,
        out_type=jax.ShapeDtypeStruct((BATCH, SEQ, FEATURES), jnp.float32),
        scratch_types=[
            pltpu.VMEM((SEQ,), jnp.int32),
            pltpu.VMEM((half, FEATURES), jnp.float32),
            pltpu.VMEM((half, FEATURES), jnp.float32),
            pltpu.SemaphoreType.DMA,
        ],
    )
    def gather(emb_hbm, idx_hbm, out_hbm, idxrow, rows0, rows1, sem):
        wid = lax.axis_index("s") * info.num_cores + lax.axis_index("c")
        b = wid // w_per_b
        off = (wid % w_per_b) * per_w
        pltpu.sync_copy(idx_hbm.at[b, 0], idxrow)
        cp0 = pltpu.async_copy(emb_hbm.at[idxrow.at[pl.ds(off, half)]],
                               rows0, sem)
        cp1 = pltpu.async_copy(emb_hbm.at[idxrow.at[pl.ds(off + half, half)]],
                               rows1, sem)
        cp0.wait()
        cp1.wait()
        pltpu.sync_copy(rows0, out_hbm.at[b, pl.ds(off, half)])
        pltpu.sync_copy(rows1, out_hbm.at[b, pl.ds(off + half, half)])

    return gather(emb, idx3)


def kernel(x, emb):
    b, n, d = x.shape
    q = x.reshape(b * n, d)
    idx3, loss11, perp11 = _distance_argmin(q, emb)
    quantized = _sc_gather(emb, idx3)
    loss = loss11.reshape(())
    perp = perp11.reshape((1,))
    return quantized, loss, idx3, perp


# speedup vs baseline: 1.0129x; 1.0075x over previous
import functools

import jax
import jax.numpy as jnp
from jax import lax
from jax.experimental import pallas as pl
from jax.experimental.pallas import tpu as pltpu
from jax.experimental.pallas import tpu_sc as plsc

CODEBOOK = 1024
FEATURES = 256
BATCH = 8
SEQ = 576
ROWS = BATCH * SEQ
BLOCK_ROWS = SEQ
NUM_BLOCKS = ROWS // BLOCK_ROWS


def _tc_body(q_hbm, kt_ref, idx_ref, loss_ref, perp_ref, cnt_ref, l2k_ref,
             qbuf, qsem):
    i = pl.program_id(0)
    kt = kt_ref[...]

    slot = lax.rem(i, 2)
    nslot = lax.rem(i + 1, 2)

    def _q_copy(blk, sl):
        return pltpu.make_async_copy(
            q_hbm.at[pl.ds(blk * BLOCK_ROWS, BLOCK_ROWS)],
            qbuf.at[sl], qsem.at[sl])

    @pl.when(i == 0)
    def _prime():
        _q_copy(0, 0).start()
        l2k_ref[...] = jnp.sum(kt * kt, axis=1, keepdims=True)

    _q_copy(i, slot).wait()

    @pl.when(i + 1 < NUM_BLOCKS)
    def _prefetch():
        _q_copy(i + 1, nslot).start()

    qb = qbuf[slot]

    sim_t = lax.dot_general(kt, qb, (((1,), (1,)), ((), ())),
                            preferred_element_type=jnp.float32)
    negdist = 2.0 * sim_t - l2k_ref[...]
    mx = jnp.max(negdist, axis=0, keepdims=True)
    eq = negdist == mx
    code_iota = lax.broadcasted_iota(
        jnp.int32, (CODEBOOK, BLOCK_ROWS), 0).astype(jnp.float32)
    cand = jnp.where(eq, code_iota, float(CODEBOOK))
    idx = jnp.min(cand, axis=0, keepdims=True).astype(jnp.int32)
    idx_ref[0, :, :] = idx

    eq_f = jnp.where(eq, 1.0, 0.0)
    cnt = lax.dot_general(eq_f, jnp.ones((BLOCK_ROWS, 1), jnp.float32),
                          (((1,), (0,)), ((), ())),
                          preferred_element_type=jnp.float32)
    block_loss = jnp.sum(qb * qb) - jnp.sum(mx)

    @pl.when(i == 0)
    def _init():
        cnt_ref[...] = cnt
        loss_ref[0, 0] = block_loss

    @pl.when(i > 0)
    def _acc():
        cnt_ref[...] += cnt
        loss_ref[0, 0] += block_loss

    @pl.when(i == NUM_BLOCKS - 1)
    def _final():
        loss_ref[0, 0] = loss_ref[0, 0] / float(ROWS * FEATURES)
        mean = cnt_ref[...] * (1.0 / ROWS)
        ent = jnp.sum(mean * jnp.log(mean + 1e-10))
        perp_ref[0, 0] = jnp.exp(-ent)


def _distance_argmin(q, emb):
    return pl.pallas_call(
        _tc_body,
        grid=(NUM_BLOCKS,),
        in_specs=[
            pl.BlockSpec(memory_space=pltpu.MemorySpace.HBM),
            pl.BlockSpec((CODEBOOK, FEATURES), lambda i: (0, 0)),
        ],
        out_specs=[
            pl.BlockSpec((1, 1, BLOCK_ROWS), lambda i: (i, 0, 0)),
            pl.BlockSpec(memory_space=pltpu.SMEM),
            pl.BlockSpec(memory_space=pltpu.SMEM),
        ],
        out_shape=[
            jax.ShapeDtypeStruct((NUM_BLOCKS, 1, BLOCK_ROWS), jnp.int32),
            jax.ShapeDtypeStruct((1, 1), jnp.float32),
            jax.ShapeDtypeStruct((1, 1), jnp.float32),
        ],
        scratch_shapes=[pltpu.VMEM((CODEBOOK, 1), jnp.float32),
                        pltpu.VMEM((CODEBOOK, 1), jnp.float32),
                        pltpu.VMEM((2, BLOCK_ROWS, FEATURES), jnp.float32),
                        pltpu.SemaphoreType.DMA((2,))],
        compiler_params=pltpu.CompilerParams(
            dimension_semantics=("arbitrary",)),
    )(q, emb)


def _sc_gather(emb, idx3):
    info = plsc.get_sparse_core_info()
    nw = info.num_cores * info.num_subcores
    per_w = ROWS // nw
    half = per_w // 2
    w_per_b = SEQ // per_w
    mesh = plsc.VectorSubcoreMesh(core_axis_name="c", subcore_axis_name="s")

    @functools.partial(
        pl.kernel, mesh=mesh,
        out_type=jax.ShapeDtypeStruct((BATCH, SEQ, FEATURES), jnp.float32),
        scratch_types=[
            pltpu.VMEM((SEQ,), jnp.int32),
            pltpu.VMEM((half, FEATURES), jnp.float32),
            pltpu.VMEM((half, FEATURES), jnp.float32),
            pltpu.SemaphoreType.DMA,
        ],
    )
    def gather(emb_hbm, idx_hbm, out_hbm, idxrow, rows0, rows1, sem):
        wid = lax.axis_index("s") * info.num_cores + lax.axis_index("c")
        b = wid // w_per_b
        off = (wid % w_per_b) * per_w
        pltpu.sync_copy(idx_hbm.at[b, 0], idxrow)
        cp0 = pltpu.async_copy(emb_hbm.at[idxrow.at[pl.ds(off, half)]],
                               rows0, sem)
        cp1 = pltpu.async_copy(emb_hbm.at[idxrow.at[pl.ds(off + half, half)]],
                               rows1, sem)
        cp0.wait()
        cp1.wait()
        pltpu.sync_copy(rows0, out_hbm.at[b, pl.ds(off, half)])
        pltpu.sync_copy(rows1, out_hbm.at[b, pl.ds(off + half, half)])

    return gather(emb, idx3)


def kernel(x, emb):
    b, n, d = x.shape
    q = x.reshape(b * n, d)
    idx3, loss11, perp11 = _distance_argmin(q, emb)
    quantized = _sc_gather(emb, idx3)
    loss = loss11.reshape(())
    perp = perp11.reshape((1,))
    return quantized, loss, idx3, perp


# speedup vs baseline: 1.0202x; 1.0072x over previous
import functools

import jax
import jax.numpy as jnp
from jax import lax
from jax.experimental import pallas as pl
from jax.experimental.pallas import tpu as pltpu
from jax.experimental.pallas import tpu_sc as plsc

CODEBOOK = 1024
FEATURES = 256
BATCH = 8
SEQ = 576
ROWS = BATCH * SEQ
BLOCK_ROWS = SEQ
NUM_BLOCKS = ROWS // BLOCK_ROWS


def _tc_body(q_ref, kt_ref, idx_ref, loss_ref, perp_ref, cnt_ref, l2k_ref):
    i = pl.program_id(0)
    qb = q_ref[...]
    kt = kt_ref[...]

    @pl.when(i == 0)
    def _prep():
        l2k_ref[...] = jnp.sum(kt * kt, axis=1, keepdims=True)

    sim_t = lax.dot_general(kt, qb, (((1,), (1,)), ((), ())),
                            preferred_element_type=jnp.float32)
    negdist = 2.0 * sim_t - l2k_ref[...]
    mx = jnp.max(negdist, axis=0, keepdims=True)
    eq = negdist == mx
    code_iota = lax.broadcasted_iota(
        jnp.int32, (CODEBOOK, BLOCK_ROWS), 0).astype(jnp.float32)
    cand = jnp.where(eq, code_iota, float(CODEBOOK))
    idx = jnp.min(cand, axis=0, keepdims=True).astype(jnp.int32)
    idx_ref[0, :, :] = idx

    eq_f = jnp.where(eq, 1.0, 0.0)
    cnt = lax.dot_general(eq_f, jnp.ones((BLOCK_ROWS, 1), jnp.float32),
                          (((1,), (0,)), ((), ())),
                          preferred_element_type=jnp.float32)
    block_loss = jnp.sum(qb * qb) - jnp.sum(mx)

    @pl.when(i == 0)
    def _init():
        cnt_ref[...] = cnt
        loss_ref[0, 0] = block_loss

    @pl.when(i > 0)
    def _acc():
        cnt_ref[...] += cnt
        loss_ref[0, 0] += block_loss

    @pl.when(i == NUM_BLOCKS - 1)
    def _final():
        loss_ref[0, 0] = loss_ref[0, 0] / float(ROWS * FEATURES)
        mean = cnt_ref[...] * (1.0 / ROWS)
        ent = jnp.sum(mean * jnp.log(mean + 1e-10))
        perp_ref[0, 0] = jnp.exp(-ent)


def _distance_argmin(q, emb):
    return pl.pallas_call(
        _tc_body,
        grid=(NUM_BLOCKS,),
        in_specs=[
            pl.BlockSpec((BLOCK_ROWS, FEATURES), lambda i: (i, 0)),
            pl.BlockSpec((CODEBOOK, FEATURES), lambda i: (0, 0)),
        ],
        out_specs=[
            pl.BlockSpec((1, 1, BLOCK_ROWS), lambda i: (i, 0, 0)),
            pl.BlockSpec(memory_space=pltpu.SMEM),
            pl.BlockSpec(memory_space=pltpu.SMEM),
        ],
        out_shape=[
            jax.ShapeDtypeStruct((NUM_BLOCKS, 1, BLOCK_ROWS), jnp.int32),
            jax.ShapeDtypeStruct((1, 1), jnp.float32),
            jax.ShapeDtypeStruct((1, 1), jnp.float32),
        ],
        scratch_shapes=[pltpu.VMEM((CODEBOOK, 1), jnp.float32),
                        pltpu.VMEM((CODEBOOK, 1), jnp.float32)],
        compiler_params=pltpu.CompilerParams(
            dimension_semantics=("arbitrary",)),
    )(q, emb)


def _sc_gather(emb, idx3):
    info = plsc.get_sparse_core_info()
    nw = info.num_cores * info.num_subcores
    per_w = ROWS // nw
    half = per_w // 2
    w_per_b = SEQ // per_w
    mesh = plsc.VectorSubcoreMesh(core_axis_name="c", subcore_axis_name="s")

    @functools.partial(
        pl.kernel, mesh=mesh,
        out_type=jax.ShapeDtypeStruct((BATCH, SEQ, FEATURES), jnp.float32),
        scratch_types=[
            pltpu.VMEM((SEQ,), jnp.int32),
            pltpu.VMEM((half, FEATURES), jnp.float32),
            pltpu.VMEM((half, FEATURES), jnp.float32),
            pltpu.SemaphoreType.DMA,
        ],
    )
    def gather(emb_hbm, idx_hbm, out_hbm, idxrow, rows0, rows1, sem):
        wid = lax.axis_index("s") * info.num_cores + lax.axis_index("c")
        b = wid // w_per_b
        off = (wid % w_per_b) * per_w
        pltpu.sync_copy(idx_hbm.at[b, 0], idxrow)
        cp0 = pltpu.async_copy(emb_hbm.at[idxrow.at[pl.ds(off, half)]],
                               rows0, sem)
        cp1 = pltpu.async_copy(emb_hbm.at[idxrow.at[pl.ds(off + half, half)]],
                               rows1, sem)
        cp0.wait()
        cp1.wait()
        pltpu.sync_copy(rows0, out_hbm.at[b, pl.ds(off, half)])
        pltpu.sync_copy(rows1, out_hbm.at[b, pl.ds(off + half, half)])

    return gather(emb, idx3)


def kernel(x, emb):
    b, n, d = x.shape
    q = x.reshape(b * n, d)
    idx3, loss11, perp11 = _distance_argmin(q, emb)
    quantized = _sc_gather(emb, idx3)
    loss = loss11.reshape(())
    perp = perp11.reshape((1,))
    return quantized, loss, idx3, perp
